# baseline (speedup 1.0000x reference)
import functools

import jax
import jax.numpy as jnp
from jax import lax
from jax.experimental import pallas as pl
from jax.experimental.pallas import tpu as pltpu
from jax.experimental.pallas import tpu_sc as plsc


@functools.lru_cache(maxsize=None)
def _build(B, S, V, D, N):
    info = plsc.get_sparse_core_info()
    NW = info.num_cores * info.num_subcores

    ROWS = B * S
    assert ROWS % NW == 0
    RPW = ROWS // NW
    C = 32
    NBUF = 2
    assert RPW % C == 0 and C <= 128
    NCH = RPW // C
    assert NCH > NBUF

    assert 16 < N <= 32
    offs = [0] + [8 * i for i in range(1, -(-(N - 16) // 8) + 1)]
    NWIN = len(offs)
    LE_ROWS = offs[-1] + 16
    assert 1 + (C - 1 - N) <= 16
    assert NBUF * C * D + LE_ROWS * D + 16 * D + RPW <= 131000

    assert S % RPW == 0 and 1 + N < C <= RPW

    WPB = S // RPW
    mesh = plsc.VectorSubcoreMesh(core_axis_name="c", subcore_axis_name="s")

    @functools.partial(
        pl.kernel,
        out_type=jax.ShapeDtypeStruct((B, S, D), jnp.float32),
        mesh=mesh,
        scratch_types=[
            pltpu.VMEM((RPW,), jnp.int32),
            [pltpu.VMEM((C, D), jnp.float32) for _ in range(NBUF)],
            pltpu.VMEM((LE_ROWS, D), jnp.float32),
            pltpu.VMEM((16, D), jnp.float32),
            [pltpu.SemaphoreType.DMA for _ in range(NBUF)],
            [pltpu.SemaphoreType.DMA for _ in range(NBUF)],
            [pltpu.SemaphoreType.DMA for _ in range(NWIN)],
            [pltpu.SemaphoreType.DMA for _ in range(NWIN)],
            pltpu.SemaphoreType.DMA,
            pltpu.SemaphoreType.DMA,
        ],
    )
    def k(tok_hbm, wte_hbm, le_hbm, out_hbm, idx_v, bufs, le_v, head_v,
          gsem, wsem, lsem, ssem, hgsem, hssem):
        wid = lax.axis_index("c") * info.num_subcores + lax.axis_index("s")
        r = wid // WPB
        cb = (wid % WPB) * RPW

        j16 = lax.iota(jnp.int32, 16)

        def le_gather(b, h):
            gidx = jnp.minimum(j16 + offs[h], N - 1)
            return pltpu.make_async_copy(
                le_hbm.at[b].at[gidx], le_v.at[pl.ds(offs[h], 16)], lsem[h]
            )

        def le_scatter(b, h):
            sidx = 1 + jnp.minimum(j16 + offs[h], N - 1)
            return pltpu.make_async_copy(
                le_v.at[pl.ds(offs[h], 16)], out_hbm.at[b].at[sidx], ssem[h]
            )

        def head_cols():
            return jnp.where(j16 == 0, 0, jnp.minimum(j16 + N, C - 1))

        def head_gather(b):
            lo = idx_v[pl.ds(0, 16)]
            hi = idx_v[pl.ds(16, 16)]
            toks = jnp.where(j16 == 0, lo[0], hi[C - 1 - 16])
            for j in range(1, C - N):
                toks = jnp.where(j16 == j, hi[N + j - 16], toks)
            return pltpu.make_async_copy(wte_hbm.at[toks], head_v, hgsem)

        def head_scatter(b):
            return pltpu.make_async_copy(
                head_v, out_hbm.at[b].at[head_cols()], hssem
            )

        def for_owner(fn):
            for b in range(B):
                owner = b * WPB

                @pl.when(wid == owner)
                def _():
                    fn(b)

        for_owner(lambda b: [le_gather(b, h).start() for h in range(NWIN)])

        pltpu.sync_copy(tok_hbm.at[r].at[pl.ds(cb, RPW)], idx_v)

        for_owner(lambda b: head_gather(b).start())

        def gather(c):
            return pltpu.make_async_copy(
                wte_hbm.at[idx_v.at[pl.ds(c * C, C)]], bufs[c % NBUF], gsem[c % NBUF]
            )

        def write(c):
            return pltpu.make_async_copy(
                bufs[c % NBUF], out_hbm.at[r, pl.ds(cb + c * C, C)], wsem[c % NBUF]
            )

        is_owner = cb == 0

        def issue(d):
            d.start()
            return d

        def splice_issue(b):
            for h in range(NWIN):
                le_gather(b, h).wait()
            for h in range(NWIN):
                le_scatter(b, h).start()
            head_gather(b).wait()
            head_scatter(b).start()

        writes = [None] * NCH
        ghs = [None] * NCH
        for c in range(NCH):
            if c >= NBUF:
                if c - NBUF == 0:
                    @pl.when(cb != 0)
                    def _():
                        write(0).wait()
                else:
                    writes[c - NBUF].wait()
            if c == 0:
                @pl.when(cb != 0)
                def _():
                    gather(0).start()
            else:
                ghs[c] = issue(gather(c))
            if c == 1:
                @pl.when(cb != 0)
                def _():
                    gather(0).wait()
                    write(0).start()
            elif c >= 2:
                ghs[c - 1].wait()
                writes[c - 1] = issue(write(c - 1))
            if c == 2:
                for_owner(splice_issue)
        ghs[NCH - 1].wait()
        writes[NCH - 1] = issue(write(NCH - 1))
        for c in range(max(1, NCH - NBUF), NCH):
            writes[c].wait()
        for_owner(lambda b: [le_scatter(b, h).wait() for h in range(NWIN)])
        for_owner(lambda b: head_scatter(b).wait())

    return k


def kernel(tokens, wte, learned_embedding):
    B, S = tokens.shape
    V, D = wte.shape
    N = learned_embedding.shape[1]
    k = _build(B, S, V, D, N)
    return k(tokens, wte, learned_embedding)
